# baseline (speedup 1.0000x reference)
import jax
import jax.numpy as jnp
from jax.experimental import pallas as pl
from jax.experimental.pallas import tpu as pltpu

_N = 320000
_D = 128
_D_OUT = 128
_S = 128
_B = 3200


def _body(xp_ref, xn_ref, bp_ref, bn_ref, w_ref, b_ref, mw_ref, mb_ref,
          out_ref, acc_p, acc_n):
    i = pl.program_id(0)
    nsteps = pl.num_programs(0)

    @pl.when(i == 0)
    def _init():
        acc_p[...] = jnp.zeros_like(acc_p)
        acc_n[...] = jnp.zeros_like(acc_n)

    w = w_ref[...].astype(jnp.bfloat16)
    bias = b_ref[...]
    iota = jax.lax.broadcasted_iota(jnp.int32, (_S, _B), 0)

    def accum(x_ref, ids_ref, acc):
        h = jnp.dot(x_ref[...].astype(jnp.bfloat16), w,
                    preferred_element_type=jnp.float32) + bias
        h = jnp.maximum(h, 0.0).astype(jnp.bfloat16)
        ids = ids_ref[0]
        onehot = (ids == iota).astype(jnp.bfloat16)
        acc[...] += jnp.dot(onehot, h, preferred_element_type=jnp.float32)

    accum(xp_ref, bp_ref, acc_p)
    accum(xn_ref, bn_ref, acc_n)

    @pl.when(i == nsteps - 1)
    def _finish():
        mw = mw_ref[...]
        out_ref[...] = (
            jnp.dot(acc_p[...], mw[:_D, :], preferred_element_type=jnp.float32)
            + jnp.dot(acc_n[...], mw[_D:, :], preferred_element_type=jnp.float32)
            + mb_ref[...]
        )


def kernel(x_p, x_n, x_p_batch, x_n_batch, emb_weight, emb_bias,
           mlp_weight, mlp_bias):
    nsteps = _N // _B
    bp = x_p_batch.reshape(nsteps, 1, _B)
    bn = x_n_batch.reshape(nsteps, 1, _B)
    return pl.pallas_call(
        _body,
        grid=(nsteps,),
        in_specs=[
            pl.BlockSpec((_B, _D), lambda i: (i, 0)),
            pl.BlockSpec((_B, _D), lambda i: (i, 0)),
            pl.BlockSpec((1, 1, _B), lambda i: (i, 0, 0)),
            pl.BlockSpec((1, 1, _B), lambda i: (i, 0, 0)),
            pl.BlockSpec((_D, _D), lambda i: (0, 0)),
            pl.BlockSpec((1, _D), lambda i: (0, 0)),
            pl.BlockSpec((2 * _D, _D_OUT), lambda i: (0, 0)),
            pl.BlockSpec((1, _D_OUT), lambda i: (0, 0)),
        ],
        out_specs=pl.BlockSpec((_S, _D_OUT), lambda i: (0, 0)),
        out_shape=jax.ShapeDtypeStruct((_S, _D_OUT), jnp.float32),
        scratch_shapes=[
            pltpu.VMEM((_S, _D), jnp.float32),
            pltpu.VMEM((_S, _D), jnp.float32),
        ],
    )(x_p, x_n, bp, bn, emb_weight, emb_bias.reshape(1, _D),
      mlp_weight, mlp_bias.reshape(1, _D_OUT))


# speedup vs baseline: 1.2343x; 1.2343x over previous
import jax
import jax.numpy as jnp
from jax.experimental import pallas as pl
from jax.experimental.pallas import tpu as pltpu

_N = 320000
_D = 128
_D_OUT = 128
_S = 128
_B = 6400


def _body(xp_ref, xn_ref, bp_ref, bn_ref, w_ref, b_ref, mw_ref, mb_ref,
          out_ref, acc_p, acc_n):
    i = pl.program_id(0)
    nsteps = pl.num_programs(0)

    @pl.when(i == 0)
    def _init():
        acc_p[...] = jnp.zeros_like(acc_p)
        acc_n[...] = jnp.zeros_like(acc_n)

    w = w_ref[...].astype(jnp.bfloat16)
    bias = b_ref[...]
    iota = jax.lax.broadcasted_iota(jnp.int32, (_S, _B), 0)

    def accum(x_ref, ids_ref, acc):
        h = jnp.dot(x_ref[...].astype(jnp.bfloat16), w,
                    preferred_element_type=jnp.float32) + bias
        h = jnp.maximum(h, 0.0).astype(jnp.bfloat16)
        ids = ids_ref[0]
        onehot = (ids == iota).astype(jnp.bfloat16)
        acc[...] += jnp.dot(onehot, h, preferred_element_type=jnp.float32)

    accum(xp_ref, bp_ref, acc_p)
    accum(xn_ref, bn_ref, acc_n)

    @pl.when(i == nsteps - 1)
    def _finish():
        mw = mw_ref[...]
        out_ref[...] = (
            jnp.dot(acc_p[...], mw[:_D, :], preferred_element_type=jnp.float32)
            + jnp.dot(acc_n[...], mw[_D:, :], preferred_element_type=jnp.float32)
            + mb_ref[...]
        )


def kernel(x_p, x_n, x_p_batch, x_n_batch, emb_weight, emb_bias,
           mlp_weight, mlp_bias):
    nsteps = _N // _B
    bp = x_p_batch.reshape(nsteps, 1, _B)
    bn = x_n_batch.reshape(nsteps, 1, _B)
    return pl.pallas_call(
        _body,
        grid=(nsteps,),
        in_specs=[
            pl.BlockSpec((_B, _D), lambda i: (i, 0)),
            pl.BlockSpec((_B, _D), lambda i: (i, 0)),
            pl.BlockSpec((1, 1, _B), lambda i: (i, 0, 0)),
            pl.BlockSpec((1, 1, _B), lambda i: (i, 0, 0)),
            pl.BlockSpec((_D, _D), lambda i: (0, 0)),
            pl.BlockSpec((1, _D), lambda i: (0, 0)),
            pl.BlockSpec((2 * _D, _D_OUT), lambda i: (0, 0)),
            pl.BlockSpec((1, _D_OUT), lambda i: (0, 0)),
        ],
        out_specs=pl.BlockSpec((_S, _D_OUT), lambda i: (0, 0)),
        out_shape=jax.ShapeDtypeStruct((_S, _D_OUT), jnp.float32),
        scratch_shapes=[
            pltpu.VMEM((_S, _D), jnp.float32),
            pltpu.VMEM((_S, _D), jnp.float32),
        ],
    )(x_p, x_n, bp, bn, emb_weight, emb_bias.reshape(1, _D),
      mlp_weight, mlp_bias.reshape(1, _D_OUT))


# speedup vs baseline: 1.3847x; 1.1218x over previous
import jax
import jax.numpy as jnp
from jax.experimental import pallas as pl
from jax.experimental.pallas import tpu as pltpu

_N = 320000
_D = 128
_D_OUT = 128
_S = 128
_B = 12800


def _body(xp_ref, xn_ref, bp_ref, bn_ref, w_ref, b_ref, mw_ref, mb_ref,
          out_ref, acc_p, acc_n):
    i = pl.program_id(0)
    nsteps = pl.num_programs(0)

    @pl.when(i == 0)
    def _init():
        acc_p[...] = jnp.zeros_like(acc_p)
        acc_n[...] = jnp.zeros_like(acc_n)

    w = w_ref[...].astype(jnp.bfloat16)
    bias = b_ref[...]
    iota = jax.lax.broadcasted_iota(jnp.int32, (_S, _B), 0)

    def accum(x_ref, ids_ref, acc):
        h = jnp.dot(x_ref[...].astype(jnp.bfloat16), w,
                    preferred_element_type=jnp.float32) + bias
        h = jnp.maximum(h, 0.0).astype(jnp.bfloat16)
        ids = ids_ref[0]
        onehot = (ids == iota).astype(jnp.bfloat16)
        acc[...] += jnp.dot(onehot, h, preferred_element_type=jnp.float32)

    accum(xp_ref, bp_ref, acc_p)
    accum(xn_ref, bn_ref, acc_n)

    @pl.when(i == nsteps - 1)
    def _finish():
        mw = mw_ref[...]
        out_ref[...] = (
            jnp.dot(acc_p[...], mw[:_D, :], preferred_element_type=jnp.float32)
            + jnp.dot(acc_n[...], mw[_D:, :], preferred_element_type=jnp.float32)
            + mb_ref[...]
        )


def kernel(x_p, x_n, x_p_batch, x_n_batch, emb_weight, emb_bias,
           mlp_weight, mlp_bias):
    nsteps = _N // _B
    bp = x_p_batch.reshape(nsteps, 1, _B)
    bn = x_n_batch.reshape(nsteps, 1, _B)
    return pl.pallas_call(
        _body,
        grid=(nsteps,),
        in_specs=[
            pl.BlockSpec((_B, _D), lambda i: (i, 0)),
            pl.BlockSpec((_B, _D), lambda i: (i, 0)),
            pl.BlockSpec((1, 1, _B), lambda i: (i, 0, 0)),
            pl.BlockSpec((1, 1, _B), lambda i: (i, 0, 0)),
            pl.BlockSpec((_D, _D), lambda i: (0, 0)),
            pl.BlockSpec((1, _D), lambda i: (0, 0)),
            pl.BlockSpec((2 * _D, _D_OUT), lambda i: (0, 0)),
            pl.BlockSpec((1, _D_OUT), lambda i: (0, 0)),
        ],
        out_specs=pl.BlockSpec((_S, _D_OUT), lambda i: (0, 0)),
        out_shape=jax.ShapeDtypeStruct((_S, _D_OUT), jnp.float32),
        scratch_shapes=[
            pltpu.VMEM((_S, _D), jnp.float32),
            pltpu.VMEM((_S, _D), jnp.float32),
        ],
    )(x_p, x_n, bp, bn, emb_weight, emb_bias.reshape(1, _D),
      mlp_weight, mlp_bias.reshape(1, _D_OUT))


# speedup vs baseline: 1.4385x; 1.0388x over previous
import jax
import jax.numpy as jnp
from jax.experimental import pallas as pl
from jax.experimental.pallas import tpu as pltpu

_N = 320000
_D = 128
_D_OUT = 128
_S = 128
_B = 16000


def _body(xp_ref, xn_ref, bp_ref, bn_ref, w_ref, b_ref, mw_ref, mb_ref,
          out_ref, acc_p, acc_n):
    i = pl.program_id(0)
    nsteps = pl.num_programs(0)

    @pl.when(i == 0)
    def _init():
        acc_p[...] = jnp.zeros_like(acc_p)
        acc_n[...] = jnp.zeros_like(acc_n)

    w = w_ref[...].astype(jnp.bfloat16)
    bias = b_ref[...]
    iota = jax.lax.broadcasted_iota(jnp.int32, (_S, _B), 0)

    def accum(x_ref, ids_ref, acc):
        h = jnp.dot(x_ref[...].astype(jnp.bfloat16), w,
                    preferred_element_type=jnp.float32) + bias
        h = jnp.maximum(h, 0.0).astype(jnp.bfloat16)
        ids = ids_ref[0]
        onehot = (ids == iota).astype(jnp.bfloat16)
        acc[...] += jnp.dot(onehot, h, preferred_element_type=jnp.float32)

    accum(xp_ref, bp_ref, acc_p)
    accum(xn_ref, bn_ref, acc_n)

    @pl.when(i == nsteps - 1)
    def _finish():
        mw = mw_ref[...]
        out_ref[...] = (
            jnp.dot(acc_p[...], mw[:_D, :], preferred_element_type=jnp.float32)
            + jnp.dot(acc_n[...], mw[_D:, :], preferred_element_type=jnp.float32)
            + mb_ref[...]
        )


def kernel(x_p, x_n, x_p_batch, x_n_batch, emb_weight, emb_bias,
           mlp_weight, mlp_bias):
    nsteps = _N // _B
    bp = x_p_batch.reshape(nsteps, 1, _B)
    bn = x_n_batch.reshape(nsteps, 1, _B)
    return pl.pallas_call(
        _body,
        grid=(nsteps,),
        in_specs=[
            pl.BlockSpec((_B, _D), lambda i: (i, 0)),
            pl.BlockSpec((_B, _D), lambda i: (i, 0)),
            pl.BlockSpec((1, 1, _B), lambda i: (i, 0, 0)),
            pl.BlockSpec((1, 1, _B), lambda i: (i, 0, 0)),
            pl.BlockSpec((_D, _D), lambda i: (0, 0)),
            pl.BlockSpec((1, _D), lambda i: (0, 0)),
            pl.BlockSpec((2 * _D, _D_OUT), lambda i: (0, 0)),
            pl.BlockSpec((1, _D_OUT), lambda i: (0, 0)),
        ],
        out_specs=pl.BlockSpec((_S, _D_OUT), lambda i: (0, 0)),
        out_shape=jax.ShapeDtypeStruct((_S, _D_OUT), jnp.float32),
        scratch_shapes=[
            pltpu.VMEM((_S, _D), jnp.float32),
            pltpu.VMEM((_S, _D), jnp.float32),
        ],
    )(x_p, x_n, bp, bn, emb_weight, emb_bias.reshape(1, _D),
      mlp_weight, mlp_bias.reshape(1, _D_OUT))


# speedup vs baseline: 1.4520x; 1.0094x over previous
import jax
import jax.numpy as jnp
from jax.experimental import pallas as pl
from jax.experimental.pallas import tpu as pltpu

_N = 320000
_D = 128
_D_OUT = 128
_S = 128
_B = 20000


def _body(xp_ref, xn_ref, bp_ref, bn_ref, w_ref, b_ref, mw_ref, mb_ref,
          out_ref, acc_p, acc_n):
    i = pl.program_id(0)
    nsteps = pl.num_programs(0)

    @pl.when(i == 0)
    def _init():
        acc_p[...] = jnp.zeros_like(acc_p)
        acc_n[...] = jnp.zeros_like(acc_n)

    w = w_ref[...].astype(jnp.bfloat16)
    bias = b_ref[...]
    iota = jax.lax.broadcasted_iota(jnp.int32, (_S, _B), 0)

    def accum(x_ref, ids_ref, acc):
        h = jnp.dot(x_ref[...].astype(jnp.bfloat16), w,
                    preferred_element_type=jnp.float32) + bias
        h = jnp.maximum(h, 0.0).astype(jnp.bfloat16)
        ids = ids_ref[0]
        onehot = (ids == iota).astype(jnp.bfloat16)
        acc[...] += jnp.dot(onehot, h, preferred_element_type=jnp.float32)

    accum(xp_ref, bp_ref, acc_p)
    accum(xn_ref, bn_ref, acc_n)

    @pl.when(i == nsteps - 1)
    def _finish():
        mw = mw_ref[...]
        out_ref[...] = (
            jnp.dot(acc_p[...], mw[:_D, :], preferred_element_type=jnp.float32)
            + jnp.dot(acc_n[...], mw[_D:, :], preferred_element_type=jnp.float32)
            + mb_ref[...]
        )


def kernel(x_p, x_n, x_p_batch, x_n_batch, emb_weight, emb_bias,
           mlp_weight, mlp_bias):
    nsteps = _N // _B
    bp = x_p_batch.reshape(nsteps, 1, _B)
    bn = x_n_batch.reshape(nsteps, 1, _B)
    return pl.pallas_call(
        _body,
        grid=(nsteps,),
        in_specs=[
            pl.BlockSpec((_B, _D), lambda i: (i, 0)),
            pl.BlockSpec((_B, _D), lambda i: (i, 0)),
            pl.BlockSpec((1, 1, _B), lambda i: (i, 0, 0)),
            pl.BlockSpec((1, 1, _B), lambda i: (i, 0, 0)),
            pl.BlockSpec((_D, _D), lambda i: (0, 0)),
            pl.BlockSpec((1, _D), lambda i: (0, 0)),
            pl.BlockSpec((2 * _D, _D_OUT), lambda i: (0, 0)),
            pl.BlockSpec((1, _D_OUT), lambda i: (0, 0)),
        ],
        out_specs=pl.BlockSpec((_S, _D_OUT), lambda i: (0, 0)),
        out_shape=jax.ShapeDtypeStruct((_S, _D_OUT), jnp.float32),
        scratch_shapes=[
            pltpu.VMEM((_S, _D), jnp.float32),
            pltpu.VMEM((_S, _D), jnp.float32),
        ],
    )(x_p, x_n, bp, bn, emb_weight, emb_bias.reshape(1, _D),
      mlp_weight, mlp_bias.reshape(1, _D_OUT))


# speedup vs baseline: 1.4911x; 1.0269x over previous
import jax
import jax.numpy as jnp
from jax.experimental import pallas as pl
from jax.experimental.pallas import tpu as pltpu

_N = 320000
_D = 128
_D_OUT = 128
_S = 128
_B = 20000
_W = 32


def _body(xp_ref, xn_ref, bp_ref, bn_ref, w_ref, b_ref, mw_ref, mb_ref,
          out_ref, acc_p, acc_n):
    i = pl.program_id(0)
    nsteps = pl.num_programs(0)

    @pl.when(i == 0)
    def _init():
        acc_p[...] = jnp.zeros_like(acc_p)
        acc_n[...] = jnp.zeros_like(acc_n)

    w = w_ref[...].astype(jnp.bfloat16)
    bias = b_ref[...]

    def accum(x_ref, ids_ref, acc):
        h = jnp.dot(x_ref[...].astype(jnp.bfloat16), w,
                    preferred_element_type=jnp.float32) + bias
        h = jnp.maximum(h, 0.0).astype(jnp.bfloat16)
        ids = ids_ref[0]
        lo = ids[0, 0]
        hi = ids[0, _B - 1]
        base = jnp.minimum((lo // 8) * 8, _S - _W)
        in_window = (hi - base) < _W

        @pl.when(in_window)
        def _fast():
            rel = ids - base
            iota_w = jax.lax.broadcasted_iota(jnp.int32, (_W, _B), 0)
            onehot = (rel == iota_w).astype(jnp.bfloat16)
            pooled = jnp.dot(onehot, h, preferred_element_type=jnp.float32)
            acc[pl.ds(base, _W), :] += pooled

        @pl.when(jnp.logical_not(in_window))
        def _slow():
            iota = jax.lax.broadcasted_iota(jnp.int32, (_S, _B), 0)
            onehot = (ids == iota).astype(jnp.bfloat16)
            acc[...] += jnp.dot(onehot, h, preferred_element_type=jnp.float32)

    accum(xp_ref, bp_ref, acc_p)
    accum(xn_ref, bn_ref, acc_n)

    @pl.when(i == nsteps - 1)
    def _finish():
        mw = mw_ref[...]
        out_ref[...] = (
            jnp.dot(acc_p[...], mw[:_D, :], preferred_element_type=jnp.float32)
            + jnp.dot(acc_n[...], mw[_D:, :], preferred_element_type=jnp.float32)
            + mb_ref[...]
        )


def kernel(x_p, x_n, x_p_batch, x_n_batch, emb_weight, emb_bias,
           mlp_weight, mlp_bias):
    nsteps = _N // _B
    bp = x_p_batch.reshape(nsteps, 1, _B)
    bn = x_n_batch.reshape(nsteps, 1, _B)
    return pl.pallas_call(
        _body,
        grid=(nsteps,),
        in_specs=[
            pl.BlockSpec((_B, _D), lambda i: (i, 0)),
            pl.BlockSpec((_B, _D), lambda i: (i, 0)),
            pl.BlockSpec((1, 1, _B), lambda i: (i, 0, 0)),
            pl.BlockSpec((1, 1, _B), lambda i: (i, 0, 0)),
            pl.BlockSpec((_D, _D), lambda i: (0, 0)),
            pl.BlockSpec((1, _D), lambda i: (0, 0)),
            pl.BlockSpec((2 * _D, _D_OUT), lambda i: (0, 0)),
            pl.BlockSpec((1, _D_OUT), lambda i: (0, 0)),
        ],
        out_specs=pl.BlockSpec((_S, _D_OUT), lambda i: (0, 0)),
        out_shape=jax.ShapeDtypeStruct((_S, _D_OUT), jnp.float32),
        scratch_shapes=[
            pltpu.VMEM((_S, _D), jnp.float32),
            pltpu.VMEM((_S, _D), jnp.float32),
        ],
    )(x_p, x_n, bp, bn, emb_weight, emb_bias.reshape(1, _D),
      mlp_weight, mlp_bias.reshape(1, _D_OUT))
